# baseline (speedup 1.0000x reference)
import functools

import jax
import jax.numpy as jnp
from jax import lax
from jax.experimental import pallas as pl
from jax.experimental.pallas import tpu as pltpu
from jax.experimental.pallas import tpu_sc as plsc

NUM_CLASSES = 1000
DIM = 128
BATCH = 16384

_info = plsc.get_sparse_core_info()
_NC, _NS = _info.num_cores, _info.num_subcores
_NW = _NC * _NS
_B_PER_W = BATCH // _NW
_NCHUNK = 4
_CHUNK = _B_PER_W // _NCHUNK
_RING = 2


@functools.partial(
    pl.kernel,
    mesh=plsc.VectorSubcoreMesh(core_axis_name="c", subcore_axis_name="s"),
    out_type=jax.ShapeDtypeStruct((BATCH, DIM), jnp.float32),
    scratch_types=[
        pltpu.VMEM((_B_PER_W,), jnp.int32),
        pltpu.VMEM((_RING, _CHUNK, DIM), jnp.float32),
        pltpu.VMEM_SHARED((NUM_CLASSES + 1, DIM), jnp.float32),
        pltpu.SemaphoreType.DMA,
        pltpu.SemaphoreType.DMA,
    ],
)
def _gather_kernel(labels_hbm, table_hbm, out_hbm, idx_v, rows_v, tab_s, gsem, ssem):
    sid = lax.axis_index("s")
    wid = sid * _NC + lax.axis_index("c")
    base = wid * _B_PER_W
    @pl.when(sid == 0)
    def _():
        pltpu.sync_copy(table_hbm, tab_s)
    pltpu.sync_copy(labels_hbm.at[pl.ds(base, _B_PER_W)], idx_v)
    plsc.subcore_barrier()
    gathers = [None] * _NCHUNK
    stores = [None] * _NCHUNK
    for c in range(_NCHUNK):
        if c >= _RING:
            stores[c - _RING].wait()
        gathers[c] = pltpu.async_copy(
            tab_s.at[idx_v.at[pl.ds(c * _CHUNK, _CHUNK)]],
            rows_v.at[c % _RING],
            gsem,
        )
        if c >= 1:
            gathers[c - 1].wait()
            stores[c - 1] = pltpu.async_copy(
                rows_v.at[(c - 1) % _RING],
                out_hbm.at[pl.ds(base + (c - 1) * _CHUNK, _CHUNK)],
                ssem,
            )
    gathers[_NCHUNK - 1].wait()
    stores[_NCHUNK - 1] = pltpu.async_copy(
        rows_v.at[(_NCHUNK - 1) % _RING],
        out_hbm.at[pl.ds(base + (_NCHUNK - 1) * _CHUNK, _CHUNK)],
        ssem,
    )
    stores[_NCHUNK - 2].wait()
    stores[_NCHUNK - 1].wait()


_KPAD = 1024
_BM = 512
_NB = BATCH // _BM


def _tc_body(lbl_ref, hi_ref, lo_ref, out_ref):
    lbl = lbl_ref[0]
    ids = lax.broadcasted_iota(jnp.int32, (_BM, _KPAD), 1)
    eq = ids == lax.broadcast_in_dim(lbl, (_BM, _KPAD), (0, 1))
    oh = jnp.where(eq, jnp.float32(1), jnp.float32(0)).astype(jnp.bfloat16)
    acc = lax.dot_general(oh, hi_ref[...], (((1,), (0,)), ((), ())),
                          preferred_element_type=jnp.float32)
    acc = acc + lax.dot_general(oh, lo_ref[...], (((1,), (0,)), ((), ())),
                                preferred_element_type=jnp.float32)
    out_ref[...] = acc


def _tc_gather(labels, table):
    hi32 = table.astype(jnp.bfloat16).astype(jnp.float32)
    hi = hi32.astype(jnp.bfloat16)
    lo = (table - hi32).astype(jnp.bfloat16)
    pad = _KPAD - table.shape[0]
    hi = jnp.pad(hi, ((0, pad), (0, 0)))
    lo = jnp.pad(lo, ((0, pad), (0, 0)))
    lbl3 = labels.reshape(_NB, _BM, 1)
    return pl.pallas_call(
        _tc_body,
        grid=(_NB,),
        in_specs=[
            pl.BlockSpec((1, _BM, 1), lambda i: (i, 0, 0)),
            pl.BlockSpec((_KPAD, DIM), lambda i: (0, 0)),
            pl.BlockSpec((_KPAD, DIM), lambda i: (0, 0)),
        ],
        out_specs=pl.BlockSpec((_BM, DIM), lambda i: (i, 0)),
        out_shape=jax.ShapeDtypeStruct((BATCH, DIM), jnp.float32),
    )(lbl3, hi, lo)


def kernel(labels, table):
    return _tc_gather(labels.astype(jnp.int32), table)


# speedup vs baseline: 1.7781x; 1.7781x over previous
import functools

import jax
import jax.numpy as jnp
from jax import lax
from jax.experimental import pallas as pl
from jax.experimental.pallas import tpu as pltpu
from jax.experimental.pallas import tpu_sc as plsc

NUM_CLASSES = 1000
DIM = 128
BATCH = 16384

_info = plsc.get_sparse_core_info()
_NC, _NS = _info.num_cores, _info.num_subcores
_NW = _NC * _NS
_B_PER_W = BATCH // _NW
_NCHUNK = 8
_CHUNK = _B_PER_W // _NCHUNK
_RING = 3


@functools.partial(
    pl.kernel,
    mesh=plsc.VectorSubcoreMesh(core_axis_name="c", subcore_axis_name="s"),
    out_type=jax.ShapeDtypeStruct((BATCH, DIM), jnp.float32),
    scratch_types=[
        pltpu.VMEM((_B_PER_W,), jnp.int32),
        pltpu.VMEM((_RING, _CHUNK, DIM), jnp.float32),
        pltpu.VMEM_SHARED((NUM_CLASSES + 1, DIM), jnp.float32),
        pltpu.SemaphoreType.DMA,
        pltpu.SemaphoreType.DMA,
    ],
)
def _gather_kernel(labels_hbm, table_hbm, out_hbm, idx_v, rows_v, tab_s, gsem, ssem):
    sid = lax.axis_index("s")
    wid = sid * _NC + lax.axis_index("c")
    base = wid * _B_PER_W
    @pl.when(sid == 0)
    def _():
        pltpu.sync_copy(table_hbm, tab_s)
    pltpu.sync_copy(labels_hbm.at[pl.ds(base, _B_PER_W)], idx_v)
    plsc.subcore_barrier()
    gathers = [None] * _NCHUNK
    stores = [None] * _NCHUNK
    for c in range(_NCHUNK):
        if c >= _RING:
            stores[c - _RING].wait()
        gathers[c] = pltpu.async_copy(
            tab_s.at[idx_v.at[pl.ds(c * _CHUNK, _CHUNK)]],
            rows_v.at[c % _RING],
            gsem,
        )
        if c >= 1:
            gathers[c - 1].wait()
            stores[c - 1] = pltpu.async_copy(
                rows_v.at[(c - 1) % _RING],
                out_hbm.at[pl.ds(base + (c - 1) * _CHUNK, _CHUNK)],
                ssem,
            )
    gathers[_NCHUNK - 1].wait()
    stores[_NCHUNK - 1] = pltpu.async_copy(
        rows_v.at[(_NCHUNK - 1) % _RING],
        out_hbm.at[pl.ds(base + (_NCHUNK - 1) * _CHUNK, _CHUNK)],
        ssem,
    )
    stores[_NCHUNK - 2].wait()
    stores[_NCHUNK - 1].wait()


def kernel(labels, table):
    return _gather_kernel(labels.astype(jnp.int32), table)
